# Initial kernel scaffold; baseline (speedup 1.0000x reference)
import jax
import jax.numpy as jnp
from jax.experimental import pallas as pl


def kernel(edge_type, embedding):
    raise NotImplementedError("write your pallas kernel here")


# speedup vs baseline: 2.1852x; 2.1852x over previous
import functools

import jax
import jax.numpy as jnp
from jax import lax
from jax.experimental import pallas as pl
from jax.experimental.pallas import tpu as pltpu, tpu_sc as plsc

N_EDGES = 320000
DIM_EMB = 128

_CHUNK = 80
_NCHUNK = N_EDGES // _CHUNK


def _make_kernel(n_workers: int):
    chunks_per_w = _NCHUNK // n_workers
    mesh = plsc.VectorSubcoreMesh(core_axis_name="c", subcore_axis_name="s")

    @functools.partial(
        pl.kernel,
        mesh=mesh,
        out_type=jax.ShapeDtypeStruct((_NCHUNK, _CHUNK, DIM_EMB), jnp.float32),
        scratch_types=[
            pltpu.VMEM((_CHUNK,), jnp.int32),
            pltpu.VMEM((_CHUNK, DIM_EMB), jnp.float32),
            pltpu.SemaphoreType.DMA,
        ],
    )
    def k(et_hbm, table_hbm, out_hbm, idx_v, rows_v, sem):
        wid = lax.axis_index("s") * 2 + lax.axis_index("c")
        base = wid * chunks_per_w

        def body(i, _):
            chunk = base + i
            pltpu.sync_copy(et_hbm.at[chunk], idx_v)
            pltpu.async_copy(table_hbm.at[idx_v], rows_v, sem).wait()
            pltpu.sync_copy(rows_v, out_hbm.at[chunk])
            return _

        lax.fori_loop(0, chunks_per_w, body, None)

    return k


def kernel(edge_type, embedding):
    et = edge_type.astype(jnp.int32).reshape(_NCHUNK, _CHUNK)
    out = _make_kernel(32)(et, embedding)
    return out.reshape(N_EDGES, DIM_EMB)


# speedup vs baseline: 2.4946x; 1.1416x over previous
import functools

import jax
import jax.numpy as jnp
from jax import lax
from jax.experimental import pallas as pl
from jax.experimental.pallas import tpu as pltpu, tpu_sc as plsc

N_EDGES = 320000
DIM_EMB = 128

_CHUNK = 80
_NCHUNK = N_EDGES // _CHUNK
_NB = 5


def _make_kernel(n_workers: int):
    cpw = _NCHUNK // n_workers
    groups = cpw // _NB
    mesh = plsc.VectorSubcoreMesh(core_axis_name="c", subcore_axis_name="s")

    @functools.partial(
        pl.kernel,
        mesh=mesh,
        out_type=jax.ShapeDtypeStruct((_NCHUNK, _CHUNK, DIM_EMB), jnp.float32),
        scratch_types=[
            pltpu.VMEM((cpw, _CHUNK), jnp.int32),
            pltpu.VMEM((_NB, _CHUNK, DIM_EMB), jnp.float32),
            *([pltpu.SemaphoreType.DMA] * _NB),
            *([pltpu.SemaphoreType.DMA] * _NB),
        ],
    )
    def k(et_hbm, table_hbm, out_hbm, idx_all, rows, *sems):
        gsems, wsems = sems[:_NB], sems[_NB:]
        wid = lax.axis_index("s") * 2 + lax.axis_index("c")
        base = wid * cpw

        pltpu.sync_copy(et_hbm.at[wid], idx_all)

        def gather(local_chunk, b):
            return pltpu.async_copy(
                table_hbm.at[idx_all.at[local_chunk]], rows.at[b], gsems[b])

        def write(local_chunk, b):
            return pltpu.async_copy(
                rows.at[b], out_hbm.at[base + local_chunk], wsems[b])

        for b in range(_NB):
            gather(b, b)

        @pl.loop(0, groups - 1)
        def grp(g):
            for b in range(_NB):
                c = g * _NB + b
                pltpu.make_async_copy(
                    table_hbm.at[idx_all.at[c]], rows.at[b], gsems[b]).wait()
                write(c, b)
            for b in range(_NB):
                c = g * _NB + b
                nxt = c + _NB
                pltpu.make_async_copy(
                    rows.at[b], out_hbm.at[base + c], wsems[b]).wait()
                gather(nxt, b)

        last = (groups - 1) * _NB
        for b in range(_NB):
            c = last + b
            pltpu.make_async_copy(
                table_hbm.at[idx_all.at[c]], rows.at[b], gsems[b]).wait()
            write(c, b)
        for b in range(_NB):
            c = last + b
            pltpu.make_async_copy(
                rows.at[b], out_hbm.at[base + c], wsems[b]).wait()

    return k


def kernel(edge_type, embedding):
    et = edge_type.astype(jnp.int32).reshape(32, _NCHUNK // 32, _CHUNK)
    out = _make_kernel(32)(et, embedding)
    return out.reshape(N_EDGES, DIM_EMB)


# speedup vs baseline: 8.7040x; 3.4892x over previous
import functools

import jax
import jax.numpy as jnp
from jax import lax
from jax.experimental import pallas as pl
from jax.experimental.pallas import tpu as pltpu, tpu_sc as plsc

N_EDGES = 320000
DIM_EMB = 128

_CHUNK = 80
_NCHUNK = N_EDGES // _CHUNK
_NB = 5


def _make_kernel(n_workers: int):
    cpw = _NCHUNK // n_workers
    groups = cpw // _NB
    mesh = plsc.VectorSubcoreMesh(core_axis_name="c", subcore_axis_name="s")

    @functools.partial(
        pl.kernel,
        mesh=mesh,
        out_type=jax.ShapeDtypeStruct((_NCHUNK, _CHUNK, DIM_EMB), jnp.float32),
        scratch_types=[
            pltpu.VMEM((cpw, _CHUNK), jnp.int32),
            pltpu.VMEM_SHARED((400, DIM_EMB), jnp.float32),
            pltpu.VMEM((_NB, _CHUNK, DIM_EMB), jnp.float32),
            *([pltpu.SemaphoreType.DMA] * _NB),
            *([pltpu.SemaphoreType.DMA] * _NB),
        ],
    )
    def k(et_hbm, table_hbm, out_hbm, idx_all, table_v, rows, *sems):
        gsems, wsems = sems[:_NB], sems[_NB:]
        wid = lax.axis_index("s") * 2 + lax.axis_index("c")
        base = wid * cpw

        pltpu.sync_copy(et_hbm.at[wid], idx_all)
        @pl.when(lax.axis_index("s") == 0)
        def _stage_table():
            pltpu.sync_copy(table_hbm, table_v)
        plsc.subcore_barrier()

        def gather(local_chunk, b):
            return pltpu.async_copy(
                table_v.at[idx_all.at[local_chunk]], rows.at[b], gsems[b])

        def write(local_chunk, b):
            return pltpu.async_copy(
                rows.at[b], out_hbm.at[base + local_chunk], wsems[b])

        for b in range(_NB):
            gather(b, b)

        @pl.loop(0, groups - 1)
        def grp(g):
            for b in range(_NB):
                c = g * _NB + b
                pltpu.make_async_copy(
                    table_v.at[idx_all.at[c]], rows.at[b], gsems[b]).wait()
                write(c, b)
            for b in range(_NB):
                c = g * _NB + b
                nxt = c + _NB
                pltpu.make_async_copy(
                    rows.at[b], out_hbm.at[base + c], wsems[b]).wait()
                gather(nxt, b)

        last = (groups - 1) * _NB
        for b in range(_NB):
            c = last + b
            pltpu.make_async_copy(
                table_v.at[idx_all.at[c]], rows.at[b], gsems[b]).wait()
            write(c, b)
        for b in range(_NB):
            c = last + b
            pltpu.make_async_copy(
                rows.at[b], out_hbm.at[base + c], wsems[b]).wait()

    return k


def kernel(edge_type, embedding):
    et = edge_type.astype(jnp.int32).reshape(32, _NCHUNK // 32, _CHUNK)
    out = _make_kernel(32)(et, embedding)
    return out.reshape(N_EDGES, DIM_EMB)


# speedup vs baseline: 8.8707x; 1.0192x over previous
import functools

import jax
import jax.numpy as jnp
from jax import lax
from jax.experimental import pallas as pl
from jax.experimental.pallas import tpu as pltpu, tpu_sc as plsc

N_EDGES = 320000
DIM_EMB = 128
DIM_DICT_ROWS = 400

_CHUNK = 80
_NCHUNK = N_EDGES // _CHUNK
_NB = 10
_LA = 5


def _make_kernel(n_workers: int):
    cpw = _NCHUNK // n_workers
    mesh = plsc.VectorSubcoreMesh(core_axis_name="c", subcore_axis_name="s")

    @functools.partial(
        pl.kernel,
        mesh=mesh,
        out_type=jax.ShapeDtypeStruct((_NCHUNK, _CHUNK, DIM_EMB), jnp.float32),
        scratch_types=[
            pltpu.VMEM((cpw, _CHUNK), jnp.int32),
            pltpu.VMEM_SHARED((DIM_DICT_ROWS, DIM_EMB), jnp.float32),
            pltpu.VMEM((_NB, _CHUNK, DIM_EMB), jnp.float32),
            *([pltpu.SemaphoreType.DMA] * _NB),
            *([pltpu.SemaphoreType.DMA] * _NB),
        ],
    )
    def k(et_hbm, table_hbm, out_hbm, idx_all, table_sh, rows, *sems):
        gsems, wsems = sems[:_NB], sems[_NB:]
        wid = lax.axis_index("s") * 2 + lax.axis_index("c")
        base = wid * cpw

        pltpu.sync_copy(et_hbm.at[wid], idx_all)

        @pl.when(lax.axis_index("s") == 0)
        def _stage_table():
            pltpu.sync_copy(table_hbm, table_sh)

        plsc.subcore_barrier()

        def gather(c, b):
            pltpu.async_copy(table_sh.at[idx_all.at[c]], rows.at[b], gsems[b])

        def wait_gather(c, b):
            pltpu.make_async_copy(
                table_sh.at[idx_all.at[c]], rows.at[b], gsems[b]).wait()

        def write(c, b):
            pltpu.async_copy(rows.at[b], out_hbm.at[base + c], wsems[b])

        def wait_write(c, b):
            pltpu.make_async_copy(
                rows.at[b], out_hbm.at[base + c], wsems[b]).wait()

        for c in range(_LA):
            gather(c, c % _NB)
        for c in range(_LA):
            wait_gather(c, c % _NB)
            write(c, c % _NB)
            gather(c + _LA, (c + _LA) % _NB)

        def step(c, k_):
            b = (_LA + k_) % _NB
            wait_gather(c, b)
            write(c, b)
            wait_write(c - _LA, (b + _LA) % _NB)
            gather(c + _LA, (b + _LA) % _NB)

        @pl.loop(0, (cpw - 2 * _LA) // _NB)
        def grp(g):
            for k_ in range(_NB):
                step(_LA + g * _NB + k_, k_)

        main_end = _LA + ((cpw - 2 * _LA) // _NB) * _NB
        for c in range(main_end, cpw - _LA):
            step(c, c - _LA)
        for c in range(cpw - _LA, cpw):
            b = c % _NB
            wait_gather(c, b)
            write(c, b)
            wait_write(c - _LA, (b + _LA) % _NB)
        for c in range(cpw - _LA, cpw):
            wait_write(c, c % _NB)

    return k


def kernel(edge_type, embedding):
    et = edge_type.astype(jnp.int32).reshape(32, _NCHUNK // 32, _CHUNK)
    out = _make_kernel(32)(et, embedding)
    return out.reshape(N_EDGES, DIM_EMB)
